# baseline (speedup 1.0000x reference)
import functools

import jax
import jax.numpy as jnp
from jax.experimental import pallas as pl
from jax.experimental.pallas import tpu as pltpu

EXPERT_CAPACITY = 1280


def _router_body(x_ref, w_ref, logits_ref, eidx_ref, pmax_ref, carry_ref,
                 tri_ref, *, blocks_per_batch, T, E):
    i = pl.program_id(0)

    @pl.when(i == 0)
    def _():
        row = jax.lax.broadcasted_iota(jnp.int32, (T, T), 0)
        col = jax.lax.broadcasted_iota(jnp.int32, (T, T), 1)
        tri_ref[...] = (row >= col).astype(jnp.float32)

    @pl.when(i % blocks_per_batch == 0)
    def _():
        carry_ref[...] = jnp.zeros_like(carry_ref)

    x = x_ref[...]
    logits = jax.lax.dot_general(
        x, w_ref[...], (((1,), (0,)), ((), ())),
        preferred_element_type=jnp.float32)
    logits_ref[...] = logits

    m = jnp.max(logits, axis=1, keepdims=True)
    e = jnp.exp(logits - m)
    s = jnp.sum(e, axis=1, keepdims=True)
    pmax_ref[...] = 1.0 / s

    eiota = jax.lax.broadcasted_iota(jnp.int32, (T, E), 1)
    idx = jnp.min(jnp.where(logits == m, eiota, E), axis=1, keepdims=True)
    oh = (eiota == idx).astype(jnp.float32)

    prio = jax.lax.dot_general(
        tri_ref[...], oh, (((1,), (0,)), ((), ())),
        preferred_element_type=jnp.float32) + carry_ref[...]
    carry_ref[...] += jnp.sum(oh, axis=0, keepdims=True)

    keep = (prio <= EXPERT_CAPACITY) & (oh > 0.0)
    eidx_ref[...] = keep.astype(jnp.int32)


def kernel(hidden_states, W):
    B, S, H = hidden_states.shape
    E = W.shape[0]
    T = 512
    x = hidden_states.reshape(B * S, H)
    wT = W.T
    nblocks = (B * S) // T
    blocks_per_batch = S // T

    logits, eidx, pmax = pl.pallas_call(
        functools.partial(_router_body, blocks_per_batch=blocks_per_batch,
                          T=T, E=E),
        grid=(nblocks,),
        in_specs=[
            pl.BlockSpec((T, H), lambda i: (i, 0)),
            pl.BlockSpec((H, E), lambda i: (0, 0)),
        ],
        out_specs=[
            pl.BlockSpec((T, E), lambda i: (i, 0)),
            pl.BlockSpec((T, E), lambda i: (i, 0)),
            pl.BlockSpec((T, 1), lambda i: (i, 0)),
        ],
        out_shape=[
            jax.ShapeDtypeStruct((B * S, E), jnp.float32),
            jax.ShapeDtypeStruct((B * S, E), jnp.int32),
            jax.ShapeDtypeStruct((B * S, 1), jnp.float32),
        ],
        scratch_shapes=[pltpu.VMEM((1, E), jnp.float32),
                        pltpu.VMEM((T, T), jnp.float32)],
        compiler_params=pltpu.CompilerParams(
            dimension_semantics=("arbitrary",)),
    )(x, wT)

    return (eidx.reshape(B, S, E),
            pmax.reshape(B, S, 1),
            logits.reshape(B, S, E))


# speedup vs baseline: 1.2241x; 1.2241x over previous
import functools

import jax
import jax.numpy as jnp
from jax.experimental import pallas as pl
from jax.experimental.pallas import tpu as pltpu

EXPERT_CAPACITY = 1280


def _router_body(x_ref, w_ref, logits_ref, eidx_ref, pmax_ref, carry_ref,
                 triu_ref, *, blocks_per_batch, T, E):
    i = pl.program_id(0)

    @pl.when(i == 0)
    def _():
        row = jax.lax.broadcasted_iota(jnp.int32, (T, T), 0)
        col = jax.lax.broadcasted_iota(jnp.int32, (T, T), 1)
        triu_ref[...] = (row <= col).astype(jnp.float32)

    @pl.when(i % blocks_per_batch == 0)
    def _():
        carry_ref[...] = jnp.zeros_like(carry_ref)

    x = x_ref[...]
    logits = jax.lax.dot_general(
        x, w_ref[...], (((1,), (0,)), ((), ())),
        preferred_element_type=jnp.float32)
    logits_ref[...] = logits

    lt = logits.T

    m = jnp.max(lt, axis=0, keepdims=True)
    s = jnp.sum(jnp.exp(lt - m), axis=0, keepdims=True)
    pmax_ref[...] = (1.0 / s).reshape(1, 1, T)

    eiota = jax.lax.broadcasted_iota(jnp.int32, (E, T), 0)
    idx = jnp.min(jnp.where(lt == m, eiota, E), axis=0, keepdims=True)
    oh = (eiota == idx).astype(jnp.float32)

    prio = jax.lax.dot_general(
        oh, triu_ref[...], (((1,), (0,)), ((), ())),
        preferred_element_type=jnp.float32)
    prio = prio + carry_ref[...]
    carry_ref[...] = prio[:, T - 1:T]

    keep = (prio <= EXPERT_CAPACITY) & (oh > 0.0)
    eidx_ref[...] = keep.astype(jnp.int32).T


def kernel(hidden_states, W):
    B, S, H = hidden_states.shape
    E = W.shape[0]
    T = 512
    x = hidden_states.reshape(B * S, H)
    wT = W.T
    nblocks = (B * S) // T
    blocks_per_batch = S // T

    logits, eidx, pmax = pl.pallas_call(
        functools.partial(_router_body, blocks_per_batch=blocks_per_batch,
                          T=T, E=E),
        grid=(nblocks,),
        in_specs=[
            pl.BlockSpec((T, H), lambda i: (i, 0)),
            pl.BlockSpec((H, E), lambda i: (0, 0)),
        ],
        out_specs=[
            pl.BlockSpec((T, E), lambda i: (i, 0)),
            pl.BlockSpec((T, E), lambda i: (i, 0)),
            pl.BlockSpec((1, 1, T), lambda i: (i, 0, 0)),
        ],
        out_shape=[
            jax.ShapeDtypeStruct((B * S, E), jnp.float32),
            jax.ShapeDtypeStruct((B * S, E), jnp.int32),
            jax.ShapeDtypeStruct((nblocks, 1, T), jnp.float32),
        ],
        scratch_shapes=[pltpu.VMEM((E, 1), jnp.float32),
                        pltpu.VMEM((T, T), jnp.float32)],
        compiler_params=pltpu.CompilerParams(
            dimension_semantics=("arbitrary",)),
    )(x, wT)

    return (eidx.reshape(B, S, E),
            pmax.reshape(B, S, 1),
            logits.reshape(B, S, E))
